# baseline (speedup 1.0000x reference)
import functools

import jax
import jax.numpy as jnp
from jax import lax
from jax.experimental import pallas as pl
from jax.experimental.pallas import tpu as pltpu
from jax.experimental.pallas import tpu_sc as plsc

ALPHA = 0.1
K_STEPS = 10

NUM_CORES = 2
NUM_SUBCORES = 16
NW = NUM_CORES * NUM_SUBCORES
CH = 128
NBUF = 2
NHALF = 2


def _cdiv(a, b):
    return (a + b - 1) // b


def _make_edge_kernel(n_pad, c, nchunk):
    rows_per_tile = n_pad // NUM_SUBCORES
    hchunk = nchunk // NHALF
    ngroups = hchunk // NBUF
    mesh = plsc.VectorSubcoreMesh(core_axis_name="c", subcore_axis_name="s")

    @functools.partial(
        pl.kernel,
        out_type=jax.ShapeDtypeStruct((NUM_CORES, n_pad, c), jnp.float32),
        mesh=mesh,
        scratch_types=[
            pltpu.VMEM_SHARED((n_pad, c), jnp.float32),
            pltpu.VMEM((hchunk, CH), jnp.int32),
            pltpu.VMEM((hchunk, CH), jnp.int32),
            pltpu.VMEM((CH, c), jnp.float32),
            pltpu.VMEM((CH, c), jnp.float32),
            pltpu.SemaphoreType.DMA,
            pltpu.SemaphoreType.DMA,
        ],
    )
    def edge_kernel(g_hbm, src_hbm, dst_hbm, z_hbm, p_hbm,
                    acc, src_vm, dst_vm, rows_a, rows_b, sem_a, sem_b):
        cid = lax.axis_index("c")
        sid = lax.axis_index("s")
        w = cid * NUM_SUBCORES + sid
        r0 = sid * rows_per_tile
        pltpu.sync_copy(z_hbm.at[pl.ds(r0, rows_per_tile)],
                        acc.at[pl.ds(r0, rows_per_tile)])
        plsc.subcore_barrier()

        for half in range(NHALF):
            pltpu.sync_copy(src_hbm.at[w, pl.ds(half * hchunk, hchunk)],
                            src_vm)
            pltpu.sync_copy(dst_hbm.at[w, pl.ds(half * hchunk, hchunk)],
                            dst_vm)
            pltpu.async_copy(g_hbm.at[src_vm.at[0]], rows_a, sem_a)

            def pair(i, carry):
                j = 2 * i
                pltpu.async_copy(g_hbm.at[src_vm.at[j + 1]], rows_b, sem_b)
                pltpu.make_async_copy(g_hbm.at[src_vm.at[j]],
                                      rows_a, sem_a).wait()
                pltpu.sync_copy(rows_a, acc.at[dst_vm.at[j]], add=True)
                pltpu.async_copy(g_hbm.at[src_vm.at[j + 2]], rows_a, sem_a)
                pltpu.make_async_copy(g_hbm.at[src_vm.at[j + 1]],
                                      rows_b, sem_b).wait()
                pltpu.sync_copy(rows_b, acc.at[dst_vm.at[j + 1]], add=True)
                return carry

            lax.fori_loop(0, hchunk // 2 - 1, pair, 0)
            jl = hchunk - 2
            pltpu.async_copy(g_hbm.at[src_vm.at[jl + 1]], rows_b, sem_b)
            pltpu.make_async_copy(g_hbm.at[src_vm.at[jl]],
                                  rows_a, sem_a).wait()
            pltpu.sync_copy(rows_a, acc.at[dst_vm.at[jl]], add=True)
            pltpu.make_async_copy(g_hbm.at[src_vm.at[jl + 1]],
                                  rows_b, sem_b).wait()
            pltpu.sync_copy(rows_b, acc.at[dst_vm.at[jl + 1]], add=True)

        plsc.subcore_barrier()
        pltpu.sync_copy(acc.at[pl.ds(r0, rows_per_tile)],
                        p_hbm.at[cid, pl.ds(r0, rows_per_tile)])

    return edge_kernel


def _mlp_call(x_pad, W1, b1, W2, b2, n_pad, f_in, hid, c, block_rows):
    grid = n_pad // block_rows

    def body(xb, w1, b1r, w2, b2r, ob):
        h1 = jnp.maximum(
            jnp.dot(xb[...], w1[...], preferred_element_type=jnp.float32)
            + b1r[...], 0.0)
        ob[...] = jnp.dot(h1, w2[...],
                          preferred_element_type=jnp.float32) + b2r[...]

    return pl.pallas_call(
        body,
        grid=(grid,),
        in_specs=[
            pl.BlockSpec((block_rows, f_in), lambda i: (i, 0)),
            pl.BlockSpec((f_in, hid), lambda i: (0, 0)),
            pl.BlockSpec((1, hid), lambda i: (0, 0)),
            pl.BlockSpec((hid, c), lambda i: (0, 0)),
            pl.BlockSpec((1, c), lambda i: (0, 0)),
        ],
        out_specs=pl.BlockSpec((block_rows, c), lambda i: (i, 0)),
        out_shape=jax.ShapeDtypeStruct((n_pad, c), jnp.float32),
    )(x_pad, W1, b1.reshape(1, hid), W2, b2.reshape(1, c))


def _prep_call(p0, p1, h, n_pad, c, block_rows):
    grid = n_pad // block_rows

    def body(p0b, p1b, hb, dinvb, gb):
        deg = p0b[:, 0:1] + p1b[:, 0:1] + 1.0
        dinv = lax.rsqrt(deg)
        dinvb[...] = dinv
        gb[...] = dinv * hb[...]

    return pl.pallas_call(
        body,
        grid=(grid,),
        in_specs=[
            pl.BlockSpec((block_rows, c), lambda i: (i, 0)),
            pl.BlockSpec((block_rows, c), lambda i: (i, 0)),
            pl.BlockSpec((block_rows, c), lambda i: (i, 0)),
        ],
        out_specs=[
            pl.BlockSpec((block_rows, 1), lambda i: (i, 0)),
            pl.BlockSpec((block_rows, c), lambda i: (i, 0)),
        ],
        out_shape=[
            jax.ShapeDtypeStruct((n_pad, 1), jnp.float32),
            jax.ShapeDtypeStruct((n_pad, c), jnp.float32),
        ],
    )(p0, p1, h)


def _combine_call(p0, p1, g, h, dinv, n_pad, c, block_rows):
    grid = n_pad // block_rows

    def body(p0b, p1b, gb, hb, dinvb, outb, gnb):
        agg = dinvb[...] * (p0b[...] + p1b[...] + gb[...])
        o = (1.0 - ALPHA) * agg + ALPHA * hb[...]
        outb[...] = o
        gnb[...] = dinvb[...] * o

    return pl.pallas_call(
        body,
        grid=(grid,),
        in_specs=[
            pl.BlockSpec((block_rows, c), lambda i: (i, 0)),
            pl.BlockSpec((block_rows, c), lambda i: (i, 0)),
            pl.BlockSpec((block_rows, c), lambda i: (i, 0)),
            pl.BlockSpec((block_rows, c), lambda i: (i, 0)),
            pl.BlockSpec((block_rows, 1), lambda i: (i, 0)),
        ],
        out_specs=[
            pl.BlockSpec((block_rows, c), lambda i: (i, 0)),
            pl.BlockSpec((block_rows, c), lambda i: (i, 0)),
        ],
        out_shape=[
            jax.ShapeDtypeStruct((n_pad, c), jnp.float32),
            jax.ShapeDtypeStruct((n_pad, c), jnp.float32),
        ],
    )(p0, p1, g, h, dinv)


def kernel(x, edge_index, W1, b1, W2, b2):
    n, f_in = x.shape
    hid = W1.shape[1]
    c = W2.shape[1]
    e = edge_index.shape[1]

    cw = 128

    n_pad = _cdiv(n + 2, 128) * 128
    nchunk = _cdiv(e, NW * CH * 2 * NHALF) * 2 * NHALF
    e_pad = NW * nchunk * CH
    block_rows = n_pad // 16

    x_pad = jnp.zeros((n_pad, f_in), jnp.float32).at[:n].set(x)
    W2p = jnp.zeros((hid, cw), jnp.float32).at[:, :c].set(W2)
    b2p = jnp.zeros((cw,), jnp.float32).at[:c].set(b2)
    src = edge_index[0]
    dst = edge_index[1]
    src3 = jnp.concatenate(
        [src, jnp.full((e_pad - e,), n + 1, jnp.int32)]).reshape(NW, nchunk, CH)
    dst3 = jnp.concatenate(
        [dst, jnp.full((e_pad - e,), n, jnp.int32)]).reshape(NW, nchunk, CH)
    zeros_feat = jnp.zeros((n_pad, cw), jnp.float32)
    ones_feat = jnp.ones((n_pad, cw), jnp.float32)

    edge_call = _make_edge_kernel(n_pad, cw, nchunk)

    h = _mlp_call(x_pad, W1, b1, W2p, b2p, n_pad, f_in, hid, cw, block_rows)

    pdeg = edge_call(ones_feat, src3, dst3, zeros_feat)
    dinv, g = _prep_call(pdeg[0], pdeg[1], h, n_pad, cw, block_rows)

    out = h
    for _ in range(K_STEPS):
        p = edge_call(g, src3, dst3, zeros_feat)
        out, g = _combine_call(p[0], p[1], g, h, dinv, n_pad, cw, block_rows)
    return out[:n, :c]


# speedup vs baseline: 2.4064x; 2.4064x over previous
import functools

import jax
import jax.numpy as jnp
from jax import lax
from jax.experimental import pallas as pl
from jax.experimental.pallas import tpu as pltpu
from jax.experimental.pallas import tpu_sc as plsc

ALPHA = 0.1
K_STEPS = 10

NUM_CORES = 2
NUM_SUBCORES = 16
NW = NUM_CORES * NUM_SUBCORES
CH = 128
NBUF = 2
NHALF = 2


def _cdiv(a, b):
    return (a + b - 1) // b


def _make_edge_kernel(n_pad, c, nchunk):
    rows_per_tile = n_pad // NUM_SUBCORES
    hchunk = nchunk // NHALF
    ngroups = hchunk // NBUF
    mesh = plsc.VectorSubcoreMesh(core_axis_name="c", subcore_axis_name="s")

    @functools.partial(
        pl.kernel,
        out_type=jax.ShapeDtypeStruct((NUM_CORES, n_pad, c), jnp.float32),
        mesh=mesh,
        scratch_types=[
            pltpu.VMEM_SHARED((n_pad, c), jnp.float32),
            pltpu.VMEM((nchunk, CH), jnp.int32),
            pltpu.VMEM((nchunk, CH), jnp.int32),
            pltpu.VMEM((CH, c), jnp.float32),
            pltpu.SemaphoreType.DMA,
        ],
        compiler_params=pltpu.CompilerParams(use_tc_tiling_on_sc=False),
    )
    def edge_kernel(g_hbm, src_hbm, dst_hbm, z_hbm, p_hbm,
                    acc, src_vm, dst_vm, rows_vm, sem):
        cid = lax.axis_index("c")
        sid = lax.axis_index("s")
        w = cid * NUM_SUBCORES + sid
        r0 = sid * rows_per_tile
        pltpu.sync_copy(z_hbm.at[pl.ds(r0, rows_per_tile)],
                        acc.at[pl.ds(r0, rows_per_tile)])
        pltpu.sync_copy(src_hbm.at[w], src_vm)
        pltpu.sync_copy(dst_hbm.at[w], dst_vm)
        plsc.subcore_barrier()

        def step(j, carry):
            pltpu.async_copy(g_hbm.at[src_vm.at[j]], rows_vm, sem).wait()
            pltpu.sync_copy(rows_vm, acc.at[dst_vm.at[j]], add=True)
            return carry

        lax.fori_loop(0, nchunk, step, 0)
        plsc.subcore_barrier()
        pltpu.sync_copy(acc.at[pl.ds(r0, rows_per_tile)],
                        p_hbm.at[cid, pl.ds(r0, rows_per_tile)])

    return edge_kernel


def _mlp_call(x_pad, W1, b1, W2, b2, n_pad, f_in, hid, c, block_rows):
    grid = n_pad // block_rows

    def body(xb, w1, b1r, w2, b2r, ob):
        h1 = jnp.maximum(
            jnp.dot(xb[...], w1[...], preferred_element_type=jnp.float32)
            + b1r[...], 0.0)
        ob[...] = jnp.dot(h1, w2[...],
                          preferred_element_type=jnp.float32) + b2r[...]

    return pl.pallas_call(
        body,
        grid=(grid,),
        in_specs=[
            pl.BlockSpec((block_rows, f_in), lambda i: (i, 0)),
            pl.BlockSpec((f_in, hid), lambda i: (0, 0)),
            pl.BlockSpec((1, hid), lambda i: (0, 0)),
            pl.BlockSpec((hid, c), lambda i: (0, 0)),
            pl.BlockSpec((1, c), lambda i: (0, 0)),
        ],
        out_specs=pl.BlockSpec((block_rows, c), lambda i: (i, 0)),
        out_shape=jax.ShapeDtypeStruct((n_pad, c), jnp.float32),
    )(x_pad, W1, b1.reshape(1, hid), W2, b2.reshape(1, c))


def _prep_call(p0, p1, h, n_pad, c, block_rows):
    grid = n_pad // block_rows

    def body(p0b, p1b, hb, dinvb, gb):
        deg = p0b[:, 0:1] + p1b[:, 0:1] + 1.0
        dinv = lax.rsqrt(deg)
        dinvb[...] = dinv
        gb[...] = dinv * hb[...]

    return pl.pallas_call(
        body,
        grid=(grid,),
        in_specs=[
            pl.BlockSpec((block_rows, c), lambda i: (i, 0)),
            pl.BlockSpec((block_rows, c), lambda i: (i, 0)),
            pl.BlockSpec((block_rows, c), lambda i: (i, 0)),
        ],
        out_specs=[
            pl.BlockSpec((block_rows, 1), lambda i: (i, 0)),
            pl.BlockSpec((block_rows, c), lambda i: (i, 0)),
        ],
        out_shape=[
            jax.ShapeDtypeStruct((n_pad, 1), jnp.float32),
            jax.ShapeDtypeStruct((n_pad, c), jnp.float32),
        ],
    )(p0, p1, h)


def _combine_call(p0, p1, g, h, dinv, n_pad, c, block_rows):
    grid = n_pad // block_rows

    def body(p0b, p1b, gb, hb, dinvb, outb, gnb):
        agg = dinvb[...] * (p0b[...] + p1b[...] + gb[...])
        o = (1.0 - ALPHA) * agg + ALPHA * hb[...]
        outb[...] = o
        gnb[...] = dinvb[...] * o

    return pl.pallas_call(
        body,
        grid=(grid,),
        in_specs=[
            pl.BlockSpec((block_rows, c), lambda i: (i, 0)),
            pl.BlockSpec((block_rows, c), lambda i: (i, 0)),
            pl.BlockSpec((block_rows, c), lambda i: (i, 0)),
            pl.BlockSpec((block_rows, c), lambda i: (i, 0)),
            pl.BlockSpec((block_rows, 1), lambda i: (i, 0)),
        ],
        out_specs=[
            pl.BlockSpec((block_rows, c), lambda i: (i, 0)),
            pl.BlockSpec((block_rows, c), lambda i: (i, 0)),
        ],
        out_shape=[
            jax.ShapeDtypeStruct((n_pad, c), jnp.float32),
            jax.ShapeDtypeStruct((n_pad, c), jnp.float32),
        ],
    )(p0, p1, g, h, dinv)


def kernel(x, edge_index, W1, b1, W2, b2):
    n, f_in = x.shape
    hid = W1.shape[1]
    c = W2.shape[1]
    e = edge_index.shape[1]

    cw = 40

    n_pad = _cdiv(n + 2, 128) * 128
    nchunk = _cdiv(e, NW * CH)
    e_pad = NW * nchunk * CH
    block_rows = n_pad // 16

    x_pad = jnp.zeros((n_pad, f_in), jnp.float32).at[:n].set(x)
    W2p = jnp.zeros((hid, cw), jnp.float32).at[:, :c].set(W2)
    b2p = jnp.zeros((cw,), jnp.float32).at[:c].set(b2)
    src = edge_index[0]
    dst = edge_index[1]
    src3 = jnp.concatenate(
        [src, jnp.full((e_pad - e,), n + 1, jnp.int32)]).reshape(NW, nchunk, CH)
    dst3 = jnp.concatenate(
        [dst, jnp.full((e_pad - e,), n, jnp.int32)]).reshape(NW, nchunk, CH)
    zeros_feat = jnp.zeros((n_pad, cw), jnp.float32)
    ones_feat = jnp.ones((n_pad, cw), jnp.float32)

    edge_call = _make_edge_kernel(n_pad, cw, nchunk)

    h = _mlp_call(x_pad, W1, b1, W2p, b2p, n_pad, f_in, hid, cw, block_rows)

    pdeg = edge_call(ones_feat, src3, dst3, zeros_feat)
    dinv, g = _prep_call(pdeg[0], pdeg[1], h, n_pad, cw, block_rows)

    out = h
    for _ in range(K_STEPS):
        p = edge_call(g, src3, dst3, zeros_feat)
        out, g = _combine_call(p[0], p[1], g, h, dinv, n_pad, cw, block_rows)
    return out[:n, :c]


# speedup vs baseline: 4.1766x; 1.7356x over previous
import functools

import jax
import jax.numpy as jnp
from jax import lax
from jax.experimental import pallas as pl
from jax.experimental.pallas import tpu as pltpu
from jax.experimental.pallas import tpu_sc as plsc

ALPHA = 0.1
K_STEPS = 10

NUM_CORES = 2
NUM_SUBCORES = 16
NW = NUM_CORES * NUM_SUBCORES
CH = 128
NBUF = 2
NHALF = 2


def _cdiv(a, b):
    return (a + b - 1) // b


def _make_edge_kernel(n_pad, c, nchunk):
    rows_per_tile = n_pad // NUM_SUBCORES
    hchunk = nchunk // NHALF
    ngroups = hchunk // NBUF
    mesh = plsc.VectorSubcoreMesh(core_axis_name="c", subcore_axis_name="s")

    @functools.partial(
        pl.kernel,
        out_type=jax.ShapeDtypeStruct((NUM_CORES, n_pad, c), jnp.float32),
        mesh=mesh,
        scratch_types=[
            pltpu.VMEM_SHARED((n_pad, c), jnp.float32),
            pltpu.VMEM_SHARED((n_pad, c), jnp.float32),
            pltpu.VMEM((nchunk, CH), jnp.int32),
            pltpu.VMEM((nchunk, CH), jnp.int32),
            pltpu.VMEM((CH, c), jnp.float32),
            pltpu.SemaphoreType.DMA,
        ],
        compiler_params=pltpu.CompilerParams(use_tc_tiling_on_sc=False),
    )
    def edge_kernel(g_hbm, src_hbm, dst_hbm, z_hbm, p_hbm,
                    acc, g_sp, src_vm, dst_vm, rows_vm, sem):
        cid = lax.axis_index("c")
        sid = lax.axis_index("s")
        w = cid * NUM_SUBCORES + sid
        r0 = sid * rows_per_tile
        pltpu.sync_copy(z_hbm.at[pl.ds(r0, rows_per_tile)],
                        acc.at[pl.ds(r0, rows_per_tile)])
        pltpu.sync_copy(g_hbm.at[pl.ds(r0, rows_per_tile)],
                        g_sp.at[pl.ds(r0, rows_per_tile)])
        pltpu.sync_copy(src_hbm.at[w], src_vm)
        pltpu.sync_copy(dst_hbm.at[w], dst_vm)
        plsc.subcore_barrier()

        def step(j, carry):
            pltpu.async_copy(g_sp.at[src_vm.at[j]], rows_vm, sem).wait()
            pltpu.sync_copy(rows_vm, acc.at[dst_vm.at[j]], add=True)
            return carry

        lax.fori_loop(0, nchunk, step, 0)
        plsc.subcore_barrier()
        pltpu.sync_copy(acc.at[pl.ds(r0, rows_per_tile)],
                        p_hbm.at[cid, pl.ds(r0, rows_per_tile)])

    return edge_kernel


def _mlp_call(x_pad, W1, b1, W2, b2, n_pad, f_in, hid, c, block_rows):
    grid = n_pad // block_rows

    def body(xb, w1, b1r, w2, b2r, ob):
        h1 = jnp.maximum(
            jnp.dot(xb[...], w1[...], preferred_element_type=jnp.float32)
            + b1r[...], 0.0)
        ob[...] = jnp.dot(h1, w2[...],
                          preferred_element_type=jnp.float32) + b2r[...]

    return pl.pallas_call(
        body,
        grid=(grid,),
        in_specs=[
            pl.BlockSpec((block_rows, f_in), lambda i: (i, 0)),
            pl.BlockSpec((f_in, hid), lambda i: (0, 0)),
            pl.BlockSpec((1, hid), lambda i: (0, 0)),
            pl.BlockSpec((hid, c), lambda i: (0, 0)),
            pl.BlockSpec((1, c), lambda i: (0, 0)),
        ],
        out_specs=pl.BlockSpec((block_rows, c), lambda i: (i, 0)),
        out_shape=jax.ShapeDtypeStruct((n_pad, c), jnp.float32),
    )(x_pad, W1, b1.reshape(1, hid), W2, b2.reshape(1, c))


def _prep_call(p0, p1, h, n_pad, c, block_rows):
    grid = n_pad // block_rows

    def body(p0b, p1b, hb, dinvb, gb):
        deg = p0b[:, 0:1] + p1b[:, 0:1] + 1.0
        dinv = lax.rsqrt(deg)
        dinvb[...] = dinv
        gb[...] = dinv * hb[...]

    return pl.pallas_call(
        body,
        grid=(grid,),
        in_specs=[
            pl.BlockSpec((block_rows, c), lambda i: (i, 0)),
            pl.BlockSpec((block_rows, c), lambda i: (i, 0)),
            pl.BlockSpec((block_rows, c), lambda i: (i, 0)),
        ],
        out_specs=[
            pl.BlockSpec((block_rows, 1), lambda i: (i, 0)),
            pl.BlockSpec((block_rows, c), lambda i: (i, 0)),
        ],
        out_shape=[
            jax.ShapeDtypeStruct((n_pad, 1), jnp.float32),
            jax.ShapeDtypeStruct((n_pad, c), jnp.float32),
        ],
    )(p0, p1, h)


def _combine_call(p0, p1, g, h, dinv, n_pad, c, block_rows):
    grid = n_pad // block_rows

    def body(p0b, p1b, gb, hb, dinvb, outb, gnb):
        agg = dinvb[...] * (p0b[...] + p1b[...] + gb[...])
        o = (1.0 - ALPHA) * agg + ALPHA * hb[...]
        outb[...] = o
        gnb[...] = dinvb[...] * o

    return pl.pallas_call(
        body,
        grid=(grid,),
        in_specs=[
            pl.BlockSpec((block_rows, c), lambda i: (i, 0)),
            pl.BlockSpec((block_rows, c), lambda i: (i, 0)),
            pl.BlockSpec((block_rows, c), lambda i: (i, 0)),
            pl.BlockSpec((block_rows, c), lambda i: (i, 0)),
            pl.BlockSpec((block_rows, 1), lambda i: (i, 0)),
        ],
        out_specs=[
            pl.BlockSpec((block_rows, c), lambda i: (i, 0)),
            pl.BlockSpec((block_rows, c), lambda i: (i, 0)),
        ],
        out_shape=[
            jax.ShapeDtypeStruct((n_pad, c), jnp.float32),
            jax.ShapeDtypeStruct((n_pad, c), jnp.float32),
        ],
    )(p0, p1, g, h, dinv)


def kernel(x, edge_index, W1, b1, W2, b2):
    n, f_in = x.shape
    hid = W1.shape[1]
    c = W2.shape[1]
    e = edge_index.shape[1]

    cw = 40

    n_pad = _cdiv(n + 2, 128) * 128
    nchunk = _cdiv(e, NW * CH)
    e_pad = NW * nchunk * CH
    block_rows = n_pad // 16

    x_pad = jnp.zeros((n_pad, f_in), jnp.float32).at[:n].set(x)
    W2p = jnp.zeros((hid, cw), jnp.float32).at[:, :c].set(W2)
    b2p = jnp.zeros((cw,), jnp.float32).at[:c].set(b2)
    src = edge_index[0]
    dst = edge_index[1]
    src3 = jnp.concatenate(
        [src, jnp.full((e_pad - e,), n + 1, jnp.int32)]).reshape(NW, nchunk, CH)
    dst3 = jnp.concatenate(
        [dst, jnp.full((e_pad - e,), n, jnp.int32)]).reshape(NW, nchunk, CH)
    zeros_feat = jnp.zeros((n_pad, cw), jnp.float32)
    ones_feat = jnp.ones((n_pad, cw), jnp.float32)

    edge_call = _make_edge_kernel(n_pad, cw, nchunk)

    h = _mlp_call(x_pad, W1, b1, W2p, b2p, n_pad, f_in, hid, cw, block_rows)

    pdeg = edge_call(ones_feat, src3, dst3, zeros_feat)
    dinv, g = _prep_call(pdeg[0], pdeg[1], h, n_pad, cw, block_rows)

    out = h
    for _ in range(K_STEPS):
        p = edge_call(g, src3, dst3, zeros_feat)
        out, g = _combine_call(p[0], p[1], g, h, dinv, n_pad, cw, block_rows)
    return out[:n, :c]


# speedup vs baseline: 4.2986x; 1.0292x over previous
import functools

import jax
import jax.numpy as jnp
from jax import lax
from jax.experimental import pallas as pl
from jax.experimental.pallas import tpu as pltpu
from jax.experimental.pallas import tpu_sc as plsc

ALPHA = 0.1
K_STEPS = 10

NUM_CORES = 2
NUM_SUBCORES = 16
NW = NUM_CORES * NUM_SUBCORES
CH = 128
NBUF = 2
NHALF = 2


def _cdiv(a, b):
    return (a + b - 1) // b


def _make_edge_kernel(n_pad, c, nchunk):
    rows_per_tile = n_pad // NUM_SUBCORES
    hchunk = nchunk // NHALF
    ngroups = hchunk // NBUF
    mesh = plsc.VectorSubcoreMesh(core_axis_name="c", subcore_axis_name="s")

    @functools.partial(
        pl.kernel,
        out_type=jax.ShapeDtypeStruct((NUM_CORES, n_pad, c), jnp.float32),
        mesh=mesh,
        scratch_types=[
            pltpu.VMEM_SHARED((n_pad, c), jnp.float32),
            pltpu.VMEM_SHARED((n_pad, c), jnp.float32),
            pltpu.VMEM((nchunk, CH), jnp.int32),
            pltpu.VMEM((nchunk, CH), jnp.int32),
            pltpu.VMEM((CH, c), jnp.float32),
            pltpu.SemaphoreType.DMA,
        ],
        compiler_params=pltpu.CompilerParams(use_tc_tiling_on_sc=False),
    )
    def edge_kernel(g_hbm, src_hbm, dst_hbm, z_hbm, p_hbm,
                    acc, g_sp, src_vm, dst_vm, rows_vm, sem):
        cid = lax.axis_index("c")
        sid = lax.axis_index("s")
        w = cid * NUM_SUBCORES + sid
        r0 = sid * rows_per_tile
        pltpu.sync_copy(z_hbm.at[pl.ds(r0, rows_per_tile)],
                        acc.at[pl.ds(r0, rows_per_tile)])
        pltpu.sync_copy(g_hbm.at[pl.ds(r0, rows_per_tile)],
                        g_sp.at[pl.ds(r0, rows_per_tile)])
        pltpu.sync_copy(src_hbm.at[w], src_vm)
        pltpu.sync_copy(dst_hbm.at[w], dst_vm)
        plsc.subcore_barrier()

        def step(j, carry):
            pltpu.async_copy(g_sp.at[src_vm.at[j]], rows_vm, sem).wait()
            pltpu.sync_copy(rows_vm, acc.at[dst_vm.at[j]], add=True)
            return carry

        lax.fori_loop(0, nchunk, step, 0)
        plsc.subcore_barrier()
        pltpu.sync_copy(acc.at[pl.ds(r0, rows_per_tile)],
                        p_hbm.at[cid, pl.ds(r0, rows_per_tile)])

    return edge_kernel


def _make_deg_kernel(n_pad, c, nchunk):
    rows_per_tile = n_pad // NUM_SUBCORES
    mesh = plsc.VectorSubcoreMesh(core_axis_name="c", subcore_axis_name="s")

    @functools.partial(
        pl.kernel,
        out_type=jax.ShapeDtypeStruct((NUM_CORES, n_pad, c), jnp.float32),
        mesh=mesh,
        scratch_types=[
            pltpu.VMEM_SHARED((n_pad, c), jnp.float32),
            pltpu.VMEM((nchunk, CH), jnp.int32),
            pltpu.VMEM((CH, c), jnp.float32),
        ],
        compiler_params=pltpu.CompilerParams(use_tc_tiling_on_sc=False),
    )
    def deg_kernel(ones_hbm, dst_hbm, z_hbm, p_hbm, acc, dst_vm, rows_vm):
        cid = lax.axis_index("c")
        sid = lax.axis_index("s")
        w = cid * NUM_SUBCORES + sid
        r0 = sid * rows_per_tile
        pltpu.sync_copy(z_hbm.at[pl.ds(r0, rows_per_tile)],
                        acc.at[pl.ds(r0, rows_per_tile)])
        pltpu.sync_copy(ones_hbm.at[pl.ds(0, CH)], rows_vm)
        pltpu.sync_copy(dst_hbm.at[w], dst_vm)
        plsc.subcore_barrier()

        def step(j, carry):
            pltpu.sync_copy(rows_vm, acc.at[dst_vm.at[j]], add=True)
            return carry

        lax.fori_loop(0, nchunk, step, 0)
        plsc.subcore_barrier()
        pltpu.sync_copy(acc.at[pl.ds(r0, rows_per_tile)],
                        p_hbm.at[cid, pl.ds(r0, rows_per_tile)])

    return deg_kernel


def _mlp_call(x_pad, W1, b1, W2, b2, n_pad, f_in, hid, c, block_rows):
    grid = n_pad // block_rows

    def body(xb, w1, b1r, w2, b2r, ob):
        h1 = jnp.maximum(
            jnp.dot(xb[...], w1[...], preferred_element_type=jnp.float32)
            + b1r[...], 0.0)
        ob[...] = jnp.dot(h1, w2[...],
                          preferred_element_type=jnp.float32) + b2r[...]

    return pl.pallas_call(
        body,
        grid=(grid,),
        in_specs=[
            pl.BlockSpec((block_rows, f_in), lambda i: (i, 0)),
            pl.BlockSpec((f_in, hid), lambda i: (0, 0)),
            pl.BlockSpec((1, hid), lambda i: (0, 0)),
            pl.BlockSpec((hid, c), lambda i: (0, 0)),
            pl.BlockSpec((1, c), lambda i: (0, 0)),
        ],
        out_specs=pl.BlockSpec((block_rows, c), lambda i: (i, 0)),
        out_shape=jax.ShapeDtypeStruct((n_pad, c), jnp.float32),
    )(x_pad, W1, b1.reshape(1, hid), W2, b2.reshape(1, c))


def _prep_call(p0, p1, h, n_pad, c, block_rows):
    grid = n_pad // block_rows

    def body(p0b, p1b, hb, dinvb, gb):
        deg = p0b[:, 0:1] + p1b[:, 0:1] + 1.0
        dinv = lax.rsqrt(deg)
        dinvb[...] = dinv
        gb[...] = dinv * hb[...]

    return pl.pallas_call(
        body,
        grid=(grid,),
        in_specs=[
            pl.BlockSpec((block_rows, c), lambda i: (i, 0)),
            pl.BlockSpec((block_rows, c), lambda i: (i, 0)),
            pl.BlockSpec((block_rows, c), lambda i: (i, 0)),
        ],
        out_specs=[
            pl.BlockSpec((block_rows, 1), lambda i: (i, 0)),
            pl.BlockSpec((block_rows, c), lambda i: (i, 0)),
        ],
        out_shape=[
            jax.ShapeDtypeStruct((n_pad, 1), jnp.float32),
            jax.ShapeDtypeStruct((n_pad, c), jnp.float32),
        ],
    )(p0, p1, h)


def _combine_call(p0, p1, g, h, dinv, n_pad, c, block_rows):
    grid = n_pad // block_rows

    def body(p0b, p1b, gb, hb, dinvb, outb, gnb):
        agg = dinvb[...] * (p0b[...] + p1b[...] + gb[...])
        o = (1.0 - ALPHA) * agg + ALPHA * hb[...]
        outb[...] = o
        gnb[...] = dinvb[...] * o

    return pl.pallas_call(
        body,
        grid=(grid,),
        in_specs=[
            pl.BlockSpec((block_rows, c), lambda i: (i, 0)),
            pl.BlockSpec((block_rows, c), lambda i: (i, 0)),
            pl.BlockSpec((block_rows, c), lambda i: (i, 0)),
            pl.BlockSpec((block_rows, c), lambda i: (i, 0)),
            pl.BlockSpec((block_rows, 1), lambda i: (i, 0)),
        ],
        out_specs=[
            pl.BlockSpec((block_rows, c), lambda i: (i, 0)),
            pl.BlockSpec((block_rows, c), lambda i: (i, 0)),
        ],
        out_shape=[
            jax.ShapeDtypeStruct((n_pad, c), jnp.float32),
            jax.ShapeDtypeStruct((n_pad, c), jnp.float32),
        ],
    )(p0, p1, g, h, dinv)


def kernel(x, edge_index, W1, b1, W2, b2):
    n, f_in = x.shape
    hid = W1.shape[1]
    c = W2.shape[1]
    e = edge_index.shape[1]

    cw = 40

    n_pad = _cdiv(n + 2, 128) * 128
    nchunk = _cdiv(e, NW * CH)
    e_pad = NW * nchunk * CH
    block_rows = n_pad // 16

    x_pad = jnp.zeros((n_pad, f_in), jnp.float32).at[:n].set(x)
    W2p = jnp.zeros((hid, cw), jnp.float32).at[:, :c].set(W2)
    b2p = jnp.zeros((cw,), jnp.float32).at[:c].set(b2)
    src = edge_index[0]
    dst = edge_index[1]
    src3 = jnp.concatenate(
        [src, jnp.full((e_pad - e,), n + 1, jnp.int32)]).reshape(NW, nchunk, CH)
    dst3 = jnp.concatenate(
        [dst, jnp.full((e_pad - e,), n, jnp.int32)]).reshape(NW, nchunk, CH)
    zeros_feat = jnp.zeros((n_pad, cw), jnp.float32)
    ones_feat = jnp.ones((n_pad, cw), jnp.float32)

    edge_call = _make_edge_kernel(n_pad, cw, nchunk)
    deg_call = _make_deg_kernel(n_pad, cw, nchunk)

    h = _mlp_call(x_pad, W1, b1, W2p, b2p, n_pad, f_in, hid, cw, block_rows)

    pdeg = deg_call(ones_feat, dst3, zeros_feat)
    dinv, g = _prep_call(pdeg[0], pdeg[1], h, n_pad, cw, block_rows)

    out = h
    for _ in range(K_STEPS):
        p = edge_call(g, src3, dst3, zeros_feat)
        out, g = _combine_call(p[0], p[1], g, h, dinv, n_pad, cw, block_rows)
    return out[:n, :c]


# speedup vs baseline: 4.3693x; 1.0165x over previous
import functools

import jax
import jax.numpy as jnp
from jax import lax
from jax.experimental import pallas as pl
from jax.experimental.pallas import tpu as pltpu
from jax.experimental.pallas import tpu_sc as plsc

ALPHA = 0.1
K_STEPS = 10

NUM_CORES = 2
NUM_SUBCORES = 16
NW = NUM_CORES * NUM_SUBCORES
CH = 128
NBUF = 2
NHALF = 2


def _cdiv(a, b):
    return (a + b - 1) // b


def _make_edge_kernel(n_pad, c, nchunk):
    rows_per_tile = n_pad // NUM_SUBCORES
    hchunk = nchunk // NHALF
    ngroups = hchunk // NBUF
    mesh = plsc.VectorSubcoreMesh(core_axis_name="c", subcore_axis_name="s")

    @functools.partial(
        pl.kernel,
        out_type=jax.ShapeDtypeStruct((NUM_CORES, n_pad, c), jnp.float32),
        mesh=mesh,
        scratch_types=[
            pltpu.VMEM_SHARED((n_pad, c), jnp.float32),
            pltpu.VMEM_SHARED((n_pad, c), jnp.float32),
            pltpu.VMEM((nchunk, CH), jnp.int32),
            pltpu.VMEM((nchunk, CH), jnp.int32),
            pltpu.VMEM((CH, c), jnp.float32),
            pltpu.SemaphoreType.DMA,
            pltpu.SemaphoreType.DMA,
            pltpu.SemaphoreType.DMA,
        ],
        compiler_params=pltpu.CompilerParams(use_tc_tiling_on_sc=False),
    )
    def edge_kernel(g_hbm, src_hbm, dst_hbm, p_hbm,
                    acc, g_sp, src_vm, dst_vm, rows_vm, sem, sem_i, sem_g):
        cid = lax.axis_index("c")
        sid = lax.axis_index("s")
        w = cid * NUM_SUBCORES + sid
        r0 = sid * rows_per_tile
        pltpu.async_copy(g_hbm.at[pl.ds(r0, rows_per_tile)],
                         acc.at[pl.ds(r0, rows_per_tile)], sem_i)
        pltpu.async_copy(g_hbm.at[pl.ds(r0, rows_per_tile)],
                         g_sp.at[pl.ds(r0, rows_per_tile)], sem_g)
        pltpu.sync_copy(src_hbm.at[w], src_vm)
        pltpu.sync_copy(dst_hbm.at[w], dst_vm)
        pltpu.make_async_copy(g_hbm.at[pl.ds(r0, rows_per_tile)],
                              acc.at[pl.ds(r0, rows_per_tile)], sem_i).wait()
        pltpu.make_async_copy(g_hbm.at[pl.ds(r0, rows_per_tile)],
                              g_sp.at[pl.ds(r0, rows_per_tile)], sem_g).wait()
        plsc.subcore_barrier()

        def step(j, carry):
            pltpu.async_copy(g_sp.at[src_vm.at[j]], rows_vm, sem).wait()
            pltpu.sync_copy(rows_vm, acc.at[dst_vm.at[j]], add=True)
            return carry

        lax.fori_loop(0, nchunk, step, 0)
        plsc.subcore_barrier()
        pltpu.sync_copy(acc.at[pl.ds(r0, rows_per_tile)],
                        p_hbm.at[cid, pl.ds(r0, rows_per_tile)])

    return edge_kernel


def _make_deg_kernel(n_pad, c, nchunk):
    rows_per_tile = n_pad // NUM_SUBCORES
    mesh = plsc.VectorSubcoreMesh(core_axis_name="c", subcore_axis_name="s")

    @functools.partial(
        pl.kernel,
        out_type=jax.ShapeDtypeStruct((NUM_CORES, n_pad, c), jnp.float32),
        mesh=mesh,
        scratch_types=[
            pltpu.VMEM_SHARED((n_pad, c), jnp.float32),
            pltpu.VMEM((nchunk, CH), jnp.int32),
            pltpu.VMEM((CH, c), jnp.float32),
        ],
        compiler_params=pltpu.CompilerParams(use_tc_tiling_on_sc=False),
    )
    def deg_kernel(ones_hbm, dst_hbm, z_hbm, p_hbm, acc, dst_vm, rows_vm):
        cid = lax.axis_index("c")
        sid = lax.axis_index("s")
        w = cid * NUM_SUBCORES + sid
        r0 = sid * rows_per_tile
        pltpu.sync_copy(z_hbm.at[pl.ds(r0, rows_per_tile)],
                        acc.at[pl.ds(r0, rows_per_tile)])
        pltpu.sync_copy(ones_hbm.at[pl.ds(0, CH)], rows_vm)
        pltpu.sync_copy(dst_hbm.at[w], dst_vm)
        plsc.subcore_barrier()

        def step(j, carry):
            pltpu.sync_copy(rows_vm, acc.at[dst_vm.at[j]], add=True)
            return carry

        lax.fori_loop(0, nchunk, step, 0)
        plsc.subcore_barrier()
        pltpu.sync_copy(acc.at[pl.ds(r0, rows_per_tile)],
                        p_hbm.at[cid, pl.ds(r0, rows_per_tile)])

    return deg_kernel


def _mlp_call(x_pad, W1, b1, W2, b2, n_pad, f_in, hid, c, block_rows):
    grid = n_pad // block_rows

    def body(xb, w1, b1r, w2, b2r, ob):
        h1 = jnp.maximum(
            jnp.dot(xb[...], w1[...], preferred_element_type=jnp.float32)
            + b1r[...], 0.0)
        ob[...] = jnp.dot(h1, w2[...],
                          preferred_element_type=jnp.float32) + b2r[...]

    return pl.pallas_call(
        body,
        grid=(grid,),
        in_specs=[
            pl.BlockSpec((block_rows, f_in), lambda i: (i, 0)),
            pl.BlockSpec((f_in, hid), lambda i: (0, 0)),
            pl.BlockSpec((1, hid), lambda i: (0, 0)),
            pl.BlockSpec((hid, c), lambda i: (0, 0)),
            pl.BlockSpec((1, c), lambda i: (0, 0)),
        ],
        out_specs=pl.BlockSpec((block_rows, c), lambda i: (i, 0)),
        out_shape=jax.ShapeDtypeStruct((n_pad, c), jnp.float32),
    )(x_pad, W1, b1.reshape(1, hid), W2, b2.reshape(1, c))


def _prep_call(p0, p1, h, n_pad, c, block_rows):
    grid = n_pad // block_rows

    def body(p0b, p1b, hb, dinvb, gb):
        deg = p0b[:, 0:1] + p1b[:, 0:1] + 1.0
        dinv = lax.rsqrt(deg)
        dinvb[...] = dinv
        gb[...] = dinv * hb[...]

    return pl.pallas_call(
        body,
        grid=(grid,),
        in_specs=[
            pl.BlockSpec((block_rows, c), lambda i: (i, 0)),
            pl.BlockSpec((block_rows, c), lambda i: (i, 0)),
            pl.BlockSpec((block_rows, c), lambda i: (i, 0)),
        ],
        out_specs=[
            pl.BlockSpec((block_rows, 1), lambda i: (i, 0)),
            pl.BlockSpec((block_rows, c), lambda i: (i, 0)),
        ],
        out_shape=[
            jax.ShapeDtypeStruct((n_pad, 1), jnp.float32),
            jax.ShapeDtypeStruct((n_pad, c), jnp.float32),
        ],
    )(p0, p1, h)


def _combine_call(p0, p1, g, h, dinv, n_pad, c, block_rows):
    grid = n_pad // block_rows

    def body(p0b, p1b, gb, hb, dinvb, outb, gnb):
        agg = dinvb[...] * (p0b[...] + p1b[...] - gb[...])
        o = (1.0 - ALPHA) * agg + ALPHA * hb[...]
        outb[...] = o
        gnb[...] = dinvb[...] * o

    return pl.pallas_call(
        body,
        grid=(grid,),
        in_specs=[
            pl.BlockSpec((block_rows, c), lambda i: (i, 0)),
            pl.BlockSpec((block_rows, c), lambda i: (i, 0)),
            pl.BlockSpec((block_rows, c), lambda i: (i, 0)),
            pl.BlockSpec((block_rows, c), lambda i: (i, 0)),
            pl.BlockSpec((block_rows, 1), lambda i: (i, 0)),
        ],
        out_specs=[
            pl.BlockSpec((block_rows, c), lambda i: (i, 0)),
            pl.BlockSpec((block_rows, c), lambda i: (i, 0)),
        ],
        out_shape=[
            jax.ShapeDtypeStruct((n_pad, c), jnp.float32),
            jax.ShapeDtypeStruct((n_pad, c), jnp.float32),
        ],
    )(p0, p1, g, h, dinv)


def kernel(x, edge_index, W1, b1, W2, b2):
    n, f_in = x.shape
    hid = W1.shape[1]
    c = W2.shape[1]
    e = edge_index.shape[1]

    cw = 40

    n_pad = _cdiv(n + 2, 128) * 128
    nchunk = _cdiv(e, NW * CH)
    e_pad = NW * nchunk * CH
    block_rows = n_pad // 16

    x_pad = jnp.zeros((n_pad, f_in), jnp.float32).at[:n].set(x)
    W2p = jnp.zeros((hid, cw), jnp.float32).at[:, :c].set(W2)
    b2p = jnp.zeros((cw,), jnp.float32).at[:c].set(b2)
    src = edge_index[0]
    dst = edge_index[1]
    src3 = jnp.concatenate(
        [src, jnp.full((e_pad - e,), n + 1, jnp.int32)]).reshape(NW, nchunk, CH)
    dst3 = jnp.concatenate(
        [dst, jnp.full((e_pad - e,), n, jnp.int32)]).reshape(NW, nchunk, CH)
    zeros_feat = jnp.zeros((n_pad, cw), jnp.float32)
    ones_feat = jnp.ones((n_pad, cw), jnp.float32)

    edge_call = _make_edge_kernel(n_pad, cw, nchunk)
    deg_call = _make_deg_kernel(n_pad, cw, nchunk)

    h = _mlp_call(x_pad, W1, b1, W2p, b2p, n_pad, f_in, hid, cw, block_rows)

    pdeg = deg_call(ones_feat, dst3, zeros_feat)
    dinv, g = _prep_call(pdeg[0], pdeg[1], h, n_pad, cw, block_rows)

    out = h
    for _ in range(K_STEPS):
        p = edge_call(g, src3, dst3)
        out, g = _combine_call(p[0], p[1], g, h, dinv, n_pad, cw, block_rows)
    return out[:n, :c]


# speedup vs baseline: 4.3699x; 1.0001x over previous
import functools

import jax
import jax.numpy as jnp
from jax import lax
from jax.experimental import pallas as pl
from jax.experimental.pallas import tpu as pltpu
from jax.experimental.pallas import tpu_sc as plsc

ALPHA = 0.1
K_STEPS = 10

NUM_CORES = 2
NUM_SUBCORES = 16
NW = NUM_CORES * NUM_SUBCORES
CH = 128


def _cdiv(a, b):
    return (a + b - 1) // b


def _make_edge_kernel(n_pad, c, nchunk):
    rows_per_tile = n_pad // NUM_SUBCORES
    mesh = plsc.VectorSubcoreMesh(core_axis_name="c", subcore_axis_name="s")

    @functools.partial(
        pl.kernel,
        out_type=jax.ShapeDtypeStruct((NUM_CORES, n_pad, c), jnp.float32),
        mesh=mesh,
        scratch_types=[
            pltpu.VMEM_SHARED((n_pad, c), jnp.float32),
            pltpu.VMEM_SHARED((n_pad, c), jnp.float32),
            pltpu.VMEM((nchunk, CH), jnp.int32),
            pltpu.VMEM((nchunk, CH), jnp.int32),
            pltpu.VMEM((CH, c), jnp.float32),
            pltpu.SemaphoreType.DMA,
            pltpu.SemaphoreType.DMA,
            pltpu.SemaphoreType.DMA,
        ],
        compiler_params=pltpu.CompilerParams(use_tc_tiling_on_sc=False),
    )
    def edge_kernel(g_hbm, src_hbm, dst_hbm, p_hbm,
                    acc, g_sp, src_vm, dst_vm, rows_vm, sem, sem_i, sem_g):
        cid = lax.axis_index("c")
        sid = lax.axis_index("s")
        w = cid * NUM_SUBCORES + sid
        r0 = sid * rows_per_tile
        pltpu.async_copy(g_hbm.at[pl.ds(r0, rows_per_tile)],
                         acc.at[pl.ds(r0, rows_per_tile)], sem_i)
        pltpu.async_copy(g_hbm.at[pl.ds(r0, rows_per_tile)],
                         g_sp.at[pl.ds(r0, rows_per_tile)], sem_g)
        pltpu.sync_copy(src_hbm.at[w], src_vm)
        pltpu.sync_copy(dst_hbm.at[w], dst_vm)
        pltpu.make_async_copy(g_hbm.at[pl.ds(r0, rows_per_tile)],
                              acc.at[pl.ds(r0, rows_per_tile)], sem_i).wait()
        pltpu.make_async_copy(g_hbm.at[pl.ds(r0, rows_per_tile)],
                              g_sp.at[pl.ds(r0, rows_per_tile)], sem_g).wait()
        plsc.subcore_barrier()

        def step(j, carry):
            pltpu.async_copy(g_sp.at[src_vm.at[j]], rows_vm, sem).wait()
            pltpu.sync_copy(rows_vm, acc.at[dst_vm.at[j]], add=True)
            return carry

        lax.fori_loop(0, nchunk, step, 0)
        plsc.subcore_barrier()
        pltpu.sync_copy(acc.at[pl.ds(r0, rows_per_tile)],
                        p_hbm.at[cid, pl.ds(r0, rows_per_tile)])

    return edge_kernel


def _make_deg_kernel(n_pad, c, nchunk):
    rows_per_tile = n_pad // NUM_SUBCORES
    mesh = plsc.VectorSubcoreMesh(core_axis_name="c", subcore_axis_name="s")

    @functools.partial(
        pl.kernel,
        out_type=jax.ShapeDtypeStruct((NUM_CORES, n_pad, c), jnp.float32),
        mesh=mesh,
        scratch_types=[
            pltpu.VMEM_SHARED((n_pad, c), jnp.float32),
            pltpu.VMEM((nchunk, CH), jnp.int32),
            pltpu.VMEM((CH, c), jnp.float32),
        ],
        compiler_params=pltpu.CompilerParams(use_tc_tiling_on_sc=False),
    )
    def deg_kernel(ones_hbm, dst_hbm, z_hbm, p_hbm, acc, dst_vm, rows_vm):
        cid = lax.axis_index("c")
        sid = lax.axis_index("s")
        w = cid * NUM_SUBCORES + sid
        r0 = sid * rows_per_tile
        pltpu.sync_copy(z_hbm.at[pl.ds(r0, rows_per_tile)],
                        acc.at[pl.ds(r0, rows_per_tile)])
        pltpu.sync_copy(ones_hbm.at[pl.ds(0, CH)], rows_vm)
        pltpu.sync_copy(dst_hbm.at[w], dst_vm)
        plsc.subcore_barrier()

        def step(j, carry):
            pltpu.sync_copy(rows_vm, acc.at[dst_vm.at[j]], add=True)
            return carry

        lax.fori_loop(0, nchunk, step, 0)
        plsc.subcore_barrier()
        pltpu.sync_copy(acc.at[pl.ds(r0, rows_per_tile)],
                        p_hbm.at[cid, pl.ds(r0, rows_per_tile)])

    return deg_kernel


def _mlp_call(x_pad, W1, b1, W2, b2, n_pad, f_in, hid, c, block_rows):
    grid = n_pad // block_rows

    def body(xb, w1, b1r, w2, b2r, ob):
        h1 = jnp.maximum(
            jnp.dot(xb[...], w1[...], preferred_element_type=jnp.float32)
            + b1r[...], 0.0)
        ob[...] = jnp.dot(h1, w2[...],
                          preferred_element_type=jnp.float32) + b2r[...]

    return pl.pallas_call(
        body,
        grid=(grid,),
        in_specs=[
            pl.BlockSpec((block_rows, f_in), lambda i: (i, 0)),
            pl.BlockSpec((f_in, hid), lambda i: (0, 0)),
            pl.BlockSpec((1, hid), lambda i: (0, 0)),
            pl.BlockSpec((hid, c), lambda i: (0, 0)),
            pl.BlockSpec((1, c), lambda i: (0, 0)),
        ],
        out_specs=pl.BlockSpec((block_rows, c), lambda i: (i, 0)),
        out_shape=jax.ShapeDtypeStruct((n_pad, c), jnp.float32),
    )(x_pad, W1, b1.reshape(1, hid), W2, b2.reshape(1, c))


def _prep_call(p0, p1, h, n_pad, c, block_rows):
    grid = n_pad // block_rows

    def body(p0b, p1b, hb, dinvb, gb):
        deg = p0b[:, 0:1] + p1b[:, 0:1] + 1.0
        dinv = lax.rsqrt(deg)
        dinvb[...] = dinv
        gb[...] = dinv * hb[...]

    return pl.pallas_call(
        body,
        grid=(grid,),
        in_specs=[
            pl.BlockSpec((block_rows, c), lambda i: (i, 0)),
            pl.BlockSpec((block_rows, c), lambda i: (i, 0)),
            pl.BlockSpec((block_rows, c), lambda i: (i, 0)),
        ],
        out_specs=[
            pl.BlockSpec((block_rows, 1), lambda i: (i, 0)),
            pl.BlockSpec((block_rows, c), lambda i: (i, 0)),
        ],
        out_shape=[
            jax.ShapeDtypeStruct((n_pad, 1), jnp.float32),
            jax.ShapeDtypeStruct((n_pad, c), jnp.float32),
        ],
    )(p0, p1, h)


def _combine_call(p0, p1, g, h, dinv, n_pad, c, block_rows):
    grid = n_pad // block_rows

    def body(p0b, p1b, gb, hb, dinvb, outb, gnb):
        agg = dinvb[...] * (p0b[...] + p1b[...] - gb[...])
        o = (1.0 - ALPHA) * agg + ALPHA * hb[...]
        outb[...] = o
        gnb[...] = dinvb[...] * o

    return pl.pallas_call(
        body,
        grid=(grid,),
        in_specs=[
            pl.BlockSpec((block_rows, c), lambda i: (i, 0)),
            pl.BlockSpec((block_rows, c), lambda i: (i, 0)),
            pl.BlockSpec((block_rows, c), lambda i: (i, 0)),
            pl.BlockSpec((block_rows, c), lambda i: (i, 0)),
            pl.BlockSpec((block_rows, 1), lambda i: (i, 0)),
        ],
        out_specs=[
            pl.BlockSpec((block_rows, c), lambda i: (i, 0)),
            pl.BlockSpec((block_rows, c), lambda i: (i, 0)),
        ],
        out_shape=[
            jax.ShapeDtypeStruct((n_pad, c), jnp.float32),
            jax.ShapeDtypeStruct((n_pad, c), jnp.float32),
        ],
    )(p0, p1, g, h, dinv)


def kernel(x, edge_index, W1, b1, W2, b2):
    n, f_in = x.shape
    hid = W1.shape[1]
    c = W2.shape[1]
    e = edge_index.shape[1]

    cw = 40

    n_pad = _cdiv(n + 2, 128) * 128
    nchunk = _cdiv(e, NW * CH)
    e_pad = NW * nchunk * CH
    block_rows = n_pad // 16

    x_pad = jnp.zeros((n_pad, f_in), jnp.float32).at[:n].set(x)
    W2p = jnp.zeros((hid, cw), jnp.float32).at[:, :c].set(W2)
    b2p = jnp.zeros((cw,), jnp.float32).at[:c].set(b2)
    src = edge_index[0]
    dst = edge_index[1]
    src3 = jnp.concatenate(
        [src, jnp.full((e_pad - e,), n + 1, jnp.int32)]).reshape(NW, nchunk, CH)
    dst3 = jnp.concatenate(
        [dst, jnp.full((e_pad - e,), n, jnp.int32)]).reshape(NW, nchunk, CH)
    zeros_feat = jnp.zeros((n_pad, cw), jnp.float32)
    ones_feat = jnp.ones((n_pad, cw), jnp.float32)

    edge_call = _make_edge_kernel(n_pad, cw, nchunk)
    deg_call = _make_deg_kernel(n_pad, cw, nchunk)

    h = _mlp_call(x_pad, W1, b1, W2p, b2p, n_pad, f_in, hid, cw, block_rows)

    pdeg = deg_call(ones_feat, dst3, zeros_feat)
    dinv, g = _prep_call(pdeg[0], pdeg[1], h, n_pad, cw, block_rows)

    out = h
    for _ in range(K_STEPS):
        p = edge_call(g, src3, dst3)
        out, g = _combine_call(p[0], p[1], g, h, dinv, n_pad, cw, block_rows)
    return out[:n, :c]
